# baseline (speedup 1.0000x reference)
import functools

import numpy as np
import jax
import jax.numpy as jnp
from jax.experimental import pallas as pl
from jax.experimental.pallas import tpu as pltpu

_B, _C, _L = 256, 20, 4096

_KS0 = 1832780943
_KS1 = 270669613

_ROTS = ((13, 15, 26, 6), (17, 29, 16, 24))


def _threefry2x32(x0, x1):
    ks = (
        jnp.uint32(_KS0),
        jnp.uint32(_KS1),
        jnp.uint32(_KS0 ^ _KS1 ^ 0x1BD11BDA),
    )
    x0 = x0 + ks[0]
    x1 = x1 + ks[1]
    for i in range(5):
        for r in _ROTS[i % 2]:
            x0 = x0 + x1
            x1 = (x1 << jnp.uint32(r)) | (x1 >> jnp.uint32(32 - r))
            x1 = x1 ^ x0
        x0 = x0 + ks[(i + 1) % 3]
        x1 = x1 + ks[(i + 2) % 3] + jnp.uint32(i + 1)
    return x0, x1


def _onehot_first_max(v, c_iota, C):
    m = jnp.max(v, axis=0, keepdims=True)
    first = jnp.min(jnp.where(v == m, c_iota, jnp.int32(C)), axis=0, keepdims=True)
    return (c_iota == first).astype(jnp.float32)


def _sample_kernel(logits_ref, out_ref, *, C, L, LB, NR):
    i = pl.program_id(0)
    j = pl.program_id(1)
    R = NR * C
    base = i * (NR * L * C) + j * (LB * C)
    r1 = jax.lax.broadcasted_iota(jnp.int32, (R, 1), 0)
    col = base + r1 + (r1 // C) * jnp.int32(L * C - C)
    row = jax.lax.broadcasted_iota(jnp.int32, (1, LB), 1) * jnp.int32(C)
    x1 = (col + row).astype(jnp.uint32)
    o0, o1 = _threefry2x32(jnp.zeros_like(x1), x1)
    bits = o0 ^ o1
    flt = jax.lax.bitcast_convert_type(
        (bits >> jnp.uint32(9)) | jnp.uint32(0x3F800000), jnp.float32
    ) - jnp.float32(1.0)
    tiny = jnp.float32(np.finfo(np.float32).tiny)
    u = jnp.maximum(flt, tiny)
    lognlogu = jnp.log(-jnp.log(u))
    c_iota = jax.lax.broadcasted_iota(jnp.int32, (C, LB), 0)
    for k in range(NR):
        v = logits_ref[k] - lognlogu[k * C : (k + 1) * C, :]
        out_ref[k] = _onehot_first_max(v, c_iota, C)


def _build(B, C, L, LB, NR, interpret=False):
    grid = (B // NR, L // LB)
    return pl.pallas_call(
        functools.partial(_sample_kernel, C=C, L=L, LB=LB, NR=NR),
        grid=grid,
        in_specs=[pl.BlockSpec((NR, C, LB), lambda i, j: (i, 0, j))],
        out_specs=pl.BlockSpec((NR, C, LB), lambda i, j: (i, 0, j)),
        out_shape=jax.ShapeDtypeStruct((B, C, L), jnp.float32),
        compiler_params=pltpu.CompilerParams(
            dimension_semantics=("parallel", "parallel")
        ),
        interpret=interpret,
    )


def kernel(logits):
    return _build(_B, _C, _L, _L, 2)(logits)


# speedup vs baseline: 1.0092x; 1.0092x over previous
import functools

import numpy as np
import jax
import jax.numpy as jnp
from jax.experimental import pallas as pl
from jax.experimental.pallas import tpu as pltpu

_B, _C, _L = 256, 20, 4096

_KS0 = 1832780943
_KS1 = 270669613

_ROTS = ((13, 15, 26, 6), (17, 29, 16, 24))


def _threefry2x32(x0, x1):
    ks = (
        jnp.uint32(_KS0),
        jnp.uint32(_KS1),
        jnp.uint32(_KS0 ^ _KS1 ^ 0x1BD11BDA),
    )
    x0 = x0 + ks[0]
    x1 = x1 + ks[1]
    for i in range(5):
        for r in _ROTS[i % 2]:
            x0 = x0 + x1
            x1 = (x1 << jnp.uint32(r)) | (x1 >> jnp.uint32(32 - r))
            x1 = x1 ^ x0
        x0 = x0 + ks[(i + 1) % 3]
        x1 = x1 + ks[(i + 2) % 3] + jnp.uint32(i + 1)
    return x0, x1


def _onehot_first_max(v, c_iota, C):
    m = jnp.max(v, axis=0, keepdims=True)
    first = jnp.min(jnp.where(v == m, c_iota, jnp.int32(C)), axis=0, keepdims=True)
    return (c_iota == first).astype(jnp.float32)


def _sample_kernel(logits_ref, out_ref, *, C, L, LB, NR):
    i = pl.program_id(0)
    j = pl.program_id(1)
    R = NR * C
    base = i * (NR * L * C) + j * (LB * C)
    r1 = jax.lax.broadcasted_iota(jnp.int32, (R, 1), 0)
    col = base + r1 + (r1 // C) * jnp.int32(L * C - C)
    row = jax.lax.broadcasted_iota(jnp.int32, (1, LB), 1) * jnp.int32(C)
    x1 = (col + row).astype(jnp.uint32)
    o0, o1 = _threefry2x32(jnp.zeros_like(x1), x1)
    bits = o0 ^ o1
    flt = jax.lax.bitcast_convert_type(
        (bits >> jnp.uint32(9)) | jnp.uint32(0x3F800000), jnp.float32
    ) - jnp.float32(1.0)
    tiny = jnp.float32(np.finfo(np.float32).tiny)
    u = jnp.maximum(flt, tiny)
    lognlogu = jnp.log(-jnp.log(u))
    c_iota = jax.lax.broadcasted_iota(jnp.int32, (C, LB), 0)
    for k in range(NR):
        v = logits_ref[k] - lognlogu[k * C : (k + 1) * C, :]
        out_ref[k] = _onehot_first_max(v, c_iota, C)


def _build(B, C, L, LB, NR, interpret=False):
    grid = (B // NR, L // LB)
    return pl.pallas_call(
        functools.partial(_sample_kernel, C=C, L=L, LB=LB, NR=NR),
        grid=grid,
        in_specs=[pl.BlockSpec((NR, C, LB), lambda i, j: (i, 0, j))],
        out_specs=pl.BlockSpec((NR, C, LB), lambda i, j: (i, 0, j)),
        out_shape=jax.ShapeDtypeStruct((B, C, L), jnp.float32),
        compiler_params=pltpu.CompilerParams(
            dimension_semantics=("parallel", "parallel")
        ),
        interpret=interpret,
    )


def kernel(logits):
    return _build(_B, _C, _L, _L, 8)(logits)
